# baseline (speedup 1.0000x reference)
import jax
import jax.numpy as jnp
from jax import lax
from jax.experimental import pallas as pl
from jax.experimental.pallas import tpu as pltpu
from jax.experimental.pallas import tpu_sc as plsc

IN_DIM = 4096
OUT_DIM = 8192
BATCH = 4096

_CBLK = 1024


def _coeff_body(wt_ref, ia_ref, ib_ref, pk_ref):
    w = wt_ref[...]
    kidx = lax.broadcasted_iota(jnp.int32, (16, _CBLK), 0)
    m = jnp.max(w, axis=0, keepdims=True)
    cand = jnp.where(w == m, kidx, 16)
    g = jnp.min(cand, axis=0, keepdims=True)
    pk_ref[...] = ia_ref[...] | (ib_ref[...] << 12) | (g << 24)


def _coeffs_from_w(W, idx_a, idx_b):
    wt = W.T
    ia3 = idx_a.reshape(OUT_DIM // _CBLK, 1, _CBLK)
    ib3 = idx_b.reshape(OUT_DIM // _CBLK, 1, _CBLK)
    pk3 = pl.pallas_call(
        _coeff_body,
        grid=(OUT_DIM // _CBLK,),
        in_specs=[
            pl.BlockSpec((16, _CBLK), lambda i: (0, i)),
            pl.BlockSpec((1, 1, _CBLK), lambda i: (i, 0, 0)),
            pl.BlockSpec((1, 1, _CBLK), lambda i: (i, 0, 0)),
        ],
        out_specs=pl.BlockSpec((1, 1, _CBLK), lambda i: (i, 0, 0)),
        out_shape=jax.ShapeDtypeStruct((OUT_DIM // _CBLK, 1, _CBLK), jnp.int32),
    )(wt, ia3, ib3)
    return pk3.reshape(OUT_DIM)


_NWORKERS = 32
_ROWS_PER_W = BATCH // _NWORKERS
_RBLK = 8
_NBLK = _ROWS_PER_W // _RBLK
_NSPLIT = 8
_JSL = OUT_DIM // _NSPLIT
_JCH = _JSL // 16


def _sc_body(x_hbm, pk_hbm, out_hbm,
             pk_v, xb0, xb1, ob0, ob1, ob2, ob3,
             sx0, sx1, so0, so1, so2, so3):
    wid = lax.axis_index("s") * 2 + lax.axis_index("c")
    base = wid * _ROWS_PER_W
    xbufs, obufs = (xb0, xb1), (ob0, ob1, ob2, ob3)
    xsems, osems = (sx0, sx1), (so0, so1, so2, so3)

    pltpu.sync_copy(pk_hbm, pk_v)

    def x_start(g, buf, sem):
        row0 = base + g * _RBLK
        for r in range(_RBLK):
            pltpu.async_copy(x_hbm.at[row0 + r],
                             buf.at[pl.ds(r * IN_DIM, IN_DIM)], sem)

    def x_wait(g, buf, sem):
        row0 = base + g * _RBLK
        for r in range(_RBLK):
            pltpu.make_async_copy(x_hbm.at[row0 + r],
                                  buf.at[pl.ds(r * IN_DIM, IN_DIM)], sem).wait()

    def out_slice(g, q):
        return out_hbm.at[pl.ds(base + g * _RBLK, _RBLK),
                          pl.ds(q * _JSL, _JSL)]

    x_start(0, xbufs[0], xsems[0])

    def outer(i, carry):
        for b in (0, 1):
            g = 2 * i + b
            x_wait(g, xbufs[b], xsems[b])

            @pl.when(g + 1 < _NBLK)
            def _():
                x_start(g + 1, xbufs[1 - b], xsems[1 - b])

            for q in range(_NSPLIT):
                ob = obufs[q % 4]
                osem = osems[q % 4]
                if q >= 4:
                    pltpu.make_async_copy(ob, out_slice(g, q - 4), osem).wait()
                else:
                    @pl.when(g >= 1)
                    def _():
                        pltpu.make_async_copy(
                            ob, out_slice(g - 1, q + _NSPLIT - 4), osem).wait()

                def jbody(jc):
                    off = q * _JSL + jc * 16
                    pk = pk_v[pl.ds(off, 16)]
                    ia = pk & 0xFFF
                    ib = (pk >> 12) & 0xFFF
                    g = pk >> 24
                    t00 = ((g >> 3) & 1).astype(jnp.float32)
                    t01 = ((g >> 2) & 1).astype(jnp.float32)
                    t10 = ((g >> 1) & 1).astype(jnp.float32)
                    t11 = (g & 1).astype(jnp.float32)
                    c0 = t00
                    c1 = t10 - t00
                    c2 = t01 - t00
                    c3 = t11 - t10 - t01 + t00
                    for r in range(0, _RBLK, 2):
                        xs0 = xbufs[b].at[pl.ds(r * IN_DIM, IN_DIM)]
                        xs1 = xbufs[b].at[pl.ds((r + 1) * IN_DIM, IN_DIM)]
                        av0 = plsc.load_gather(xs0, [ia])
                        bv0 = plsc.load_gather(xs0, [ib])
                        av1 = plsc.load_gather(xs1, [ia])
                        bv1 = plsc.load_gather(xs1, [ib])
                        ob[r, pl.ds(jc * 16, 16)] = (
                            (c0 + av0 * c1) + bv0 * (c2 + av0 * c3))
                        ob[r + 1, pl.ds(jc * 16, 16)] = (
                            (c0 + av1 * c1) + bv1 * (c2 + av1 * c3))

                plsc.parallel_loop(0, _JCH, 1, unroll=4)(jbody)
                pltpu.async_copy(ob, out_slice(g, q), osem)
        return carry

    lax.fori_loop(0, _NBLK // 2, outer, 0)
    for q in range(_NSPLIT - 4, _NSPLIT):
        pltpu.make_async_copy(obufs[q % 4], out_slice(_NBLK - 1, q),
                              osems[q % 4]).wait()


def _sc_call(x, pk):
    mesh = plsc.VectorSubcoreMesh(core_axis_name="c", subcore_axis_name="s")
    run = pl.kernel(
        _sc_body,
        out_type=jax.ShapeDtypeStruct((BATCH, OUT_DIM), jnp.float32),
        mesh=mesh,
        scratch_types=[
            pltpu.VMEM((OUT_DIM,), jnp.int32),
            pltpu.VMEM((_RBLK * IN_DIM,), jnp.float32),
            pltpu.VMEM((_RBLK * IN_DIM,), jnp.float32),
            pltpu.VMEM((_RBLK, _JSL), jnp.float32),
            pltpu.VMEM((_RBLK, _JSL), jnp.float32),
            pltpu.VMEM((_RBLK, _JSL), jnp.float32),
            pltpu.VMEM((_RBLK, _JSL), jnp.float32),
            pltpu.SemaphoreType.DMA,
            pltpu.SemaphoreType.DMA,
            pltpu.SemaphoreType.DMA,
            pltpu.SemaphoreType.DMA,
            pltpu.SemaphoreType.DMA,
            pltpu.SemaphoreType.DMA,
        ],
        compiler_params=pltpu.CompilerParams(needs_layout_passes=False),
    )
    return run(x, pk)


def kernel(x, W, idx_a, idx_b):
    pk = _coeffs_from_w(W, idx_a, idx_b)
    return _sc_call(x, pk)


# speedup vs baseline: 1.0790x; 1.0790x over previous
import jax
import jax.numpy as jnp
from jax import lax
from jax.experimental import pallas as pl
from jax.experimental.pallas import tpu as pltpu
from jax.experimental.pallas import tpu_sc as plsc

IN_DIM = 4096
OUT_DIM = 8192
BATCH = 4096

_CBLK = 1024


def _coeff_body(wt_ref, ia_ref, ib_ref, pk_ref):
    w = wt_ref[...]
    kidx = lax.broadcasted_iota(jnp.int32, (16, _CBLK), 0)
    m = jnp.max(w, axis=0, keepdims=True)
    cand = jnp.where(w == m, kidx, 16)
    g = jnp.min(cand, axis=0, keepdims=True)
    pk_ref[...] = ia_ref[...] | (ib_ref[...] << 12) | (g << 24)


def _coeffs_from_w(W, idx_a, idx_b):
    wt = W.T
    ia3 = idx_a.reshape(OUT_DIM // _CBLK, 1, _CBLK)
    ib3 = idx_b.reshape(OUT_DIM // _CBLK, 1, _CBLK)
    pk3 = pl.pallas_call(
        _coeff_body,
        grid=(OUT_DIM // _CBLK,),
        in_specs=[
            pl.BlockSpec((16, _CBLK), lambda i: (0, i)),
            pl.BlockSpec((1, 1, _CBLK), lambda i: (i, 0, 0)),
            pl.BlockSpec((1, 1, _CBLK), lambda i: (i, 0, 0)),
        ],
        out_specs=pl.BlockSpec((1, 1, _CBLK), lambda i: (i, 0, 0)),
        out_shape=jax.ShapeDtypeStruct((OUT_DIM // _CBLK, 1, _CBLK), jnp.int32),
    )(wt, ia3, ib3)
    return pk3.reshape(OUT_DIM)


_NWORKERS = 32
_ROWS_PER_W = BATCH // _NWORKERS
_RBLK = 8
_NBLK = _ROWS_PER_W // _RBLK
_NSPLIT = 8
_JSL = OUT_DIM // _NSPLIT
_JCH = _JSL // 16


def _sc_body(x_hbm, pk_hbm, out_hbm,
             pk_v, xb0, xb1, ob0, ob1, ob2, ob3,
             sx0, sx1, so0, so1, so2, so3):
    wid = lax.axis_index("s") * 2 + lax.axis_index("c")
    base = wid * _ROWS_PER_W
    xbufs, obufs = (xb0, xb1), (ob0, ob1, ob2, ob3)
    xsems, osems = (sx0, sx1), (so0, so1, so2, so3)

    pltpu.sync_copy(pk_hbm, pk_v)

    def x_start(g, buf, sem):
        row0 = base + g * _RBLK
        for r in range(_RBLK):
            pltpu.async_copy(x_hbm.at[row0 + r],
                             buf.at[pl.ds(r * IN_DIM, IN_DIM)], sem)

    def x_wait(g, buf, sem):
        row0 = base + g * _RBLK
        for r in range(_RBLK):
            pltpu.make_async_copy(x_hbm.at[row0 + r],
                                  buf.at[pl.ds(r * IN_DIM, IN_DIM)], sem).wait()

    def out_slice(g, q):
        return out_hbm.at[pl.ds(base + g * _RBLK, _RBLK),
                          pl.ds(q * _JSL, _JSL)]

    x_start(0, xbufs[0], xsems[0])

    def outer(i, carry):
        for b in (0, 1):
            g = 2 * i + b
            x_wait(g, xbufs[b], xsems[b])

            @pl.when(g + 1 < _NBLK)
            def _():
                x_start(g + 1, xbufs[1 - b], xsems[1 - b])

            for q in range(_NSPLIT):
                ob = obufs[q % 4]
                osem = osems[q % 4]
                if q >= 4:
                    pltpu.make_async_copy(ob, out_slice(g, q - 4), osem).wait()
                else:
                    @pl.when(g >= 1)
                    def _():
                        pltpu.make_async_copy(
                            ob, out_slice(g - 1, q + _NSPLIT - 4), osem).wait()

                def jbody(jc):
                    off = q * _JSL + jc * 16
                    pk = pk_v[pl.ds(off, 16)]
                    ia = pk & 0xFFF
                    ib = (pk >> 12) & 0xFFF
                    g = pk >> 24
                    t00 = ((g >> 3) & 1).astype(jnp.float32)
                    t01 = ((g >> 2) & 1).astype(jnp.float32)
                    t10 = ((g >> 1) & 1).astype(jnp.float32)
                    t11 = (g & 1).astype(jnp.float32)
                    c0 = t00
                    c1 = t10 - t00
                    c2 = t01 - t00
                    c3 = t11 - t10 - t01 + t00
                    for r in range(0, _RBLK, 2):
                        xs0 = xbufs[b].at[pl.ds(r * IN_DIM, IN_DIM)]
                        xs1 = xbufs[b].at[pl.ds((r + 1) * IN_DIM, IN_DIM)]
                        av0 = plsc.load_gather(xs0, [ia])
                        bv0 = plsc.load_gather(xs0, [ib])
                        av1 = plsc.load_gather(xs1, [ia])
                        bv1 = plsc.load_gather(xs1, [ib])
                        ob[r, pl.ds(jc * 16, 16)] = (
                            (c0 + av0 * c1) + bv0 * (c2 + av0 * c3))
                        ob[r + 1, pl.ds(jc * 16, 16)] = (
                            (c0 + av1 * c1) + bv1 * (c2 + av1 * c3))

                plsc.parallel_loop(0, _JCH, 1, unroll=2)(jbody)
                pltpu.async_copy(ob, out_slice(g, q), osem)
        return carry

    lax.fori_loop(0, _NBLK // 2, outer, 0)
    for q in range(_NSPLIT - 4, _NSPLIT):
        pltpu.make_async_copy(obufs[q % 4], out_slice(_NBLK - 1, q),
                              osems[q % 4]).wait()


def _sc_call(x, pk):
    mesh = plsc.VectorSubcoreMesh(core_axis_name="c", subcore_axis_name="s")
    run = pl.kernel(
        _sc_body,
        out_type=jax.ShapeDtypeStruct((BATCH, OUT_DIM), jnp.float32),
        mesh=mesh,
        scratch_types=[
            pltpu.VMEM((OUT_DIM,), jnp.int32),
            pltpu.VMEM((_RBLK * IN_DIM,), jnp.float32),
            pltpu.VMEM((_RBLK * IN_DIM,), jnp.float32),
            pltpu.VMEM((_RBLK, _JSL), jnp.float32),
            pltpu.VMEM((_RBLK, _JSL), jnp.float32),
            pltpu.VMEM((_RBLK, _JSL), jnp.float32),
            pltpu.VMEM((_RBLK, _JSL), jnp.float32),
            pltpu.SemaphoreType.DMA,
            pltpu.SemaphoreType.DMA,
            pltpu.SemaphoreType.DMA,
            pltpu.SemaphoreType.DMA,
            pltpu.SemaphoreType.DMA,
            pltpu.SemaphoreType.DMA,
        ],
        compiler_params=pltpu.CompilerParams(needs_layout_passes=False),
    )
    return run(x, pk)


def kernel(x, W, idx_a, idx_b):
    pk = _coeffs_from_w(W, idx_a, idx_b)
    return _sc_call(x, pk)
